# baseline (speedup 1.0000x reference)
import functools

import jax
import jax.numpy as jnp
import numpy as np
from jax.experimental import pallas as pl
from jax.experimental.pallas import tpu as pltpu

D_MODEL = 1024
MEM_SIZE = 4096
N_QUBITS = 16
N_BLOCKS = 8
NBITS = 4

def _addr_kernel(x_ref, wa_ref, ba_ref, aw_ref, lw_ref, x8_ref, *,
                 rows_per_blk, t_write):
    i = pl.program_id(0)
    rb = rows_per_blk

    qq_t = jax.lax.dot_general(wa_ref[...], x_ref[...], (((0,), (1,)), ((), ())),
                               preferred_element_type=jnp.float32)
    qq_t = qq_t + ba_ref[...]
    p1 = jnp.where(qq_t > 0, jnp.sin(qq_t / 2.0) ** 2, 0.0)

    jsub = jax.lax.broadcasted_iota(jnp.int32, (N_BLOCKS, rb), 0)
    s = None
    c = None
    for q in range(NBITS):
        bit = ((jsub >> q) & 1).astype(jnp.float32)
        bm = 2.0 * bit - 1.0
        term = p1[q:q + 1, :] * bm
        s = term if s is None else s + term
        c = (1.0 - bit) if c is None else c + (1.0 - bit)
    s = s + c

    m = jnp.max(s, axis=0, keepdims=True)
    e = jnp.exp(s - m)
    aw_ref[...] = e / jnp.sum(e, axis=0, keepdims=True)

    i0 = jnp.min(jnp.where(s == m, jsub, N_BLOCKS), axis=0, keepdims=True)
    oh0 = jsub == i0
    s1 = jnp.where(oh0, -jnp.inf, s)
    m1 = jnp.max(s1, axis=0, keepdims=True)
    i1 = jnp.min(jnp.where(s1 == m1, jsub, N_BLOCKS), axis=0, keepdims=True)
    oh = oh0 | (jsub == i1)

    t = i * rb + jax.lax.broadcasted_iota(jnp.int32, (N_BLOCKS, rb), 1)
    cand = jnp.where(oh & (t < t_write), t, -1)
    part = jnp.max(cand, axis=1, keepdims=True)

    @pl.when(i == 0)
    def _():
        lw_ref[...] = jnp.full(lw_ref.shape, -1, jnp.int32)
        x8_ref[...] = jnp.zeros(x8_ref.shape, jnp.float32)

    @pl.when(i * rb < t_write)
    def _():
        cur = lw_ref[...]
        for j in range(N_BLOCKS):
            tj = part[j, 0]
            pred = tj > cur[j, 0]
            rel = jnp.maximum(tj - i * rb, 0)
            row = x_ref[pl.ds(rel, 1), :]
            x8_ref[pl.ds(j, 1), :] = jnp.where(pred, row,
                                               x8_ref[pl.ds(j, 1), :])

    lw_ref[...] = jnp.maximum(lw_ref[...], jnp.broadcast_to(part, lw_ref.shape))


def _factor_kernel(lw_ref, x8_ref, md_ref, wc_ref, bc_ref, wo_ref, bo_ref,
                   wq_ref, wk_ref, wv_ref, wout_ref, wg1_ref,
                   g_ref, m8_ref, g8_ref):
    cont = jnp.dot(x8_ref[...], wc_ref[...], preferred_element_type=jnp.float32)
    cont = cont + bc_ref[...]
    rows = []
    for j in range(N_BLOCKS):
        rows.append(jnp.where(lw_ref[j] >= 0, cont[j:j + 1, :],
                              md_ref[j:j + 1, :]))
    mem = jnp.concatenate(rows, axis=0)
    v8 = jnp.dot(mem, wo_ref[...], preferred_element_type=jnp.float32)
    v8 = v8 + bo_ref[...]
    vq = jnp.dot(v8, wq_ref[...], preferred_element_type=jnp.float32)
    vk = jnp.dot(v8, wk_ref[...], preferred_element_type=jnp.float32)
    vv = jnp.dot(v8, wv_ref[...], preferred_element_type=jnp.float32)
    g_ref[...] = jax.lax.dot_general(
        vq, vk, (((1,), (1,)), ((), ())),
        preferred_element_type=jnp.float32) / jnp.sqrt(jnp.float32(D_MODEL))
    m8 = jnp.dot(vv, wout_ref[...], preferred_element_type=jnp.float32)
    m8_ref[...] = m8
    g8_ref[...] = jnp.dot(m8, wg1_ref[...], preferred_element_type=jnp.float32)


def _attn_kernel(af_ref, ab_ref, g_ref, m8_ref, g8_ref, x_ref, wg2_ref,
                 bg_ref, o_ref):
    a_full_t = af_ref[...]
    a_blk_t = ab_ref[...]
    h_t = jax.lax.dot_general(g_ref[...], a_blk_t, (((0,), (0,)), ((), ())),
                              preferred_element_type=jnp.float32)
    sc = jax.lax.dot_general(h_t, a_full_t, (((0,), (0,)), ((), ())),
                             preferred_element_type=jnp.float32)
    m = jnp.max(sc, axis=1, keepdims=True)
    e = jnp.exp(sc - m)
    pa = jax.lax.dot_general(e, a_full_t, (((1,), (1,)), ((), ())),
                             preferred_element_type=jnp.float32)
    pa = pa / jnp.sum(e, axis=1, keepdims=True)
    att = jnp.dot(pa, m8_ref[...], preferred_element_type=jnp.float32)
    x = x_ref[0]
    glin = jnp.dot(pa, g8_ref[...], preferred_element_type=jnp.float32)
    glin = glin + jnp.dot(x, wg2_ref[...], preferred_element_type=jnp.float32)
    glin = glin + bg_ref[...]
    g = jax.nn.sigmoid(glin)
    o_ref[0] = g * att + (1.0 - g) * x


def kernel(query, input_data, memory_data, W_addr, b_addr, W_content, b_content,
           W_out, b_out, Wq, Wk, Wv, Wo, W_gate, b_gate):
    B, S, D = input_data.shape
    T = B * S
    xin = input_data.reshape(T, D)
    xq = query.reshape(T, D)
    x_all = jnp.concatenate([xin, xq], axis=0)

    RB = 1024
    n_blk = (2 * T) // RB
    aw, lw8, x8 = pl.pallas_call(
        functools.partial(_addr_kernel, rows_per_blk=RB, t_write=T),
        grid=(n_blk,),
        in_specs=[
            pl.BlockSpec((RB, D), lambda i: (i, 0)),
            pl.BlockSpec((D, N_QUBITS), lambda i: (0, 0)),
            pl.BlockSpec((N_QUBITS, 1), lambda i: (0, 0)),
        ],
        out_specs=[
            pl.BlockSpec((N_BLOCKS, RB), lambda i: (0, i)),
            pl.BlockSpec((N_BLOCKS, N_BLOCKS), lambda i: (0, 0)),
            pl.BlockSpec((N_BLOCKS, D), lambda i: (0, 0)),
        ],
        out_shape=[
            jax.ShapeDtypeStruct((N_BLOCKS, 2 * T), jnp.float32),
            jax.ShapeDtypeStruct((N_BLOCKS, N_BLOCKS), jnp.int32),
            jax.ShapeDtypeStruct((N_BLOCKS, D), jnp.float32),
        ],
    )(x_all, W_addr, b_addr.reshape(N_QUBITS, 1))
    lw = lw8[:, 0]

    G, M8, G8 = pl.pallas_call(
        _factor_kernel,
        in_specs=[pl.BlockSpec(memory_space=pltpu.SMEM)]
        + [pl.BlockSpec(memory_space=pltpu.VMEM)] * 11,
        out_shape=[
            jax.ShapeDtypeStruct((N_BLOCKS, N_BLOCKS), jnp.float32),
            jax.ShapeDtypeStruct((N_BLOCKS, D), jnp.float32),
            jax.ShapeDtypeStruct((N_BLOCKS, D), jnp.float32),
        ],
    )(lw, x8, memory_data, W_content, b_content.reshape(1, MEM_SIZE),
      W_out, b_out.reshape(1, D), Wq, Wk, Wv, Wo, W_gate[:D])

    aw_rt = aw[:, T:]
    BS = 512
    n_s = S // BS
    out = pl.pallas_call(
        _attn_kernel,
        grid=(B, n_s),
        in_specs=[
            pl.BlockSpec((N_BLOCKS, S), lambda b, s: (0, b)),
            pl.BlockSpec((N_BLOCKS, BS), lambda b, s: (0, b * n_s + s)),
            pl.BlockSpec((N_BLOCKS, N_BLOCKS), lambda b, s: (0, 0)),
            pl.BlockSpec((N_BLOCKS, D), lambda b, s: (0, 0)),
            pl.BlockSpec((N_BLOCKS, D), lambda b, s: (0, 0)),
            pl.BlockSpec((1, BS, D), lambda b, s: (b, s, 0)),
            pl.BlockSpec((D, D), lambda b, s: (0, 0)),
            pl.BlockSpec((1, D), lambda b, s: (0, 0)),
        ],
        out_specs=pl.BlockSpec((1, BS, D), lambda b, s: (b, s, 0)),
        out_shape=jax.ShapeDtypeStruct((B, S, D), jnp.float32),
    )(aw_rt, aw_rt, G, M8, G8, input_data, W_gate[D:], b_gate.reshape(1, D))
    return out


# speedup vs baseline: 1.2837x; 1.2837x over previous
import functools

import jax
import jax.numpy as jnp
import numpy as np
from jax.experimental import pallas as pl
from jax.experimental.pallas import tpu as pltpu

D_MODEL = 1024
MEM_SIZE = 4096
N_QUBITS = 16
N_BLOCKS = 8
NBITS = 4

def _scores_t(x_ref, wa_ref, ba_ref, rb):
    qq_t = jax.lax.dot_general(wa_ref[...], x_ref[...], (((0,), (1,)), ((), ())),
                               preferred_element_type=jnp.float32)
    qq_t = qq_t + ba_ref[...]
    p1 = jnp.where(qq_t > 0, jnp.sin(qq_t / 2.0) ** 2, 0.0)

    jsub = jax.lax.broadcasted_iota(jnp.int32, (N_BLOCKS, rb), 0)
    s = None
    c = None
    for q in range(NBITS):
        bit = ((jsub >> q) & 1).astype(jnp.float32)
        bm = 2.0 * bit - 1.0
        term = p1[q:q + 1, :] * bm
        s = term if s is None else s + term
        c = (1.0 - bit) if c is None else c + (1.0 - bit)
    return s + c, jsub


def _write_kernel(x_ref, wa_ref, ba_ref, lw_ref, x8_ref, *, rows_per_blk):
    i = pl.program_id(0)
    rb = rows_per_blk
    s, jsub = _scores_t(x_ref, wa_ref, ba_ref, rb)

    m = jnp.max(s, axis=0, keepdims=True)
    i0 = jnp.min(jnp.where(s == m, jsub, N_BLOCKS), axis=0, keepdims=True)
    oh0 = jsub == i0
    s1 = jnp.where(oh0, -jnp.inf, s)
    m1 = jnp.max(s1, axis=0, keepdims=True)
    i1 = jnp.min(jnp.where(s1 == m1, jsub, N_BLOCKS), axis=0, keepdims=True)
    oh = oh0 | (jsub == i1)

    t = i * rb + jax.lax.broadcasted_iota(jnp.int32, (N_BLOCKS, rb), 1)
    cand = jnp.where(oh, t, -1)
    part = jnp.max(cand, axis=1, keepdims=True)

    @pl.when(i == 0)
    def _():
        lw_ref[...] = jnp.full(lw_ref.shape, -1, jnp.int32)
        x8_ref[...] = jnp.zeros(x8_ref.shape, jnp.float32)

    cur = lw_ref[...]
    for j in range(N_BLOCKS):
        tj = part[j, 0]
        pred = tj > cur[j, 0]
        rel = jnp.maximum(tj - i * rb, 0)
        row = x_ref[pl.ds(rel, 1), :]
        x8_ref[pl.ds(j, 1), :] = jnp.where(pred, row, x8_ref[pl.ds(j, 1), :])

    lw_ref[...] = jnp.maximum(lw_ref[...], jnp.broadcast_to(part, lw_ref.shape))


def _read_kernel(x_ref, wa_ref, ba_ref, aw_ref, *, rows_per_blk):
    s, _ = _scores_t(x_ref, wa_ref, ba_ref, rows_per_blk)
    m = jnp.max(s, axis=0, keepdims=True)
    e = jnp.exp(s - m)
    aw_ref[...] = e / jnp.sum(e, axis=0, keepdims=True)


def _factor_kernel(lw_ref, x8_ref, md_ref, wc_ref, bc_ref, wo_ref, bo_ref,
                   wq_ref, wk_ref, wv_ref, wout_ref, wg1_ref,
                   g_ref, m8_ref, g8_ref):
    cont = jnp.dot(x8_ref[...], wc_ref[...], preferred_element_type=jnp.float32)
    cont = cont + bc_ref[...]
    rows = []
    for j in range(N_BLOCKS):
        rows.append(jnp.where(lw_ref[j] >= 0, cont[j:j + 1, :],
                              md_ref[j:j + 1, :]))
    mem = jnp.concatenate(rows, axis=0)
    v8 = jnp.dot(mem, wo_ref[...], preferred_element_type=jnp.float32)
    v8 = v8 + bo_ref[...]
    vq = jnp.dot(v8, wq_ref[...], preferred_element_type=jnp.float32)
    vk = jnp.dot(v8, wk_ref[...], preferred_element_type=jnp.float32)
    vv = jnp.dot(v8, wv_ref[...], preferred_element_type=jnp.float32)
    g_ref[...] = jax.lax.dot_general(
        vq, vk, (((1,), (1,)), ((), ())),
        preferred_element_type=jnp.float32) / jnp.sqrt(jnp.float32(D_MODEL))
    m8 = jnp.dot(vv, wout_ref[...], preferred_element_type=jnp.float32)
    m8_ref[...] = m8
    g8_ref[...] = jnp.dot(m8, wg1_ref[...], preferred_element_type=jnp.float32)


def _attn_kernel(af_ref, ab_ref, g_ref, m8_ref, g8_ref, x_ref, wg2_ref,
                 bg_ref, o_ref):
    a_full_t = af_ref[...]
    a_blk_t = ab_ref[...]
    h_t = jax.lax.dot_general(g_ref[...], a_blk_t, (((0,), (0,)), ((), ())),
                              preferred_element_type=jnp.float32)
    sc = jax.lax.dot_general(h_t, a_full_t, (((0,), (0,)), ((), ())),
                             preferred_element_type=jnp.float32)
    m = jnp.max(sc, axis=1, keepdims=True)
    e = jnp.exp(sc - m)
    pa = jax.lax.dot_general(e, a_full_t, (((1,), (1,)), ((), ())),
                             preferred_element_type=jnp.float32)
    pa = pa / jnp.sum(e, axis=1, keepdims=True)
    att = jnp.dot(pa, m8_ref[...], preferred_element_type=jnp.float32)
    x = x_ref[0]
    glin = jnp.dot(pa, g8_ref[...], preferred_element_type=jnp.float32)
    glin = glin + jnp.dot(x.astype(jnp.bfloat16), wg2_ref[...],
                          preferred_element_type=jnp.float32)
    glin = glin + bg_ref[...]
    g = jax.nn.sigmoid(glin)
    o_ref[0] = g * att + (1.0 - g) * x


def kernel(query, input_data, memory_data, W_addr, b_addr, W_content, b_content,
           W_out, b_out, Wq, Wk, Wv, Wo, W_gate, b_gate):
    B, S, D = input_data.shape
    T = B * S
    xin = input_data.reshape(T, D)
    xq = query.reshape(T, D)

    RB = 1024
    lw8, x8 = pl.pallas_call(
        functools.partial(_write_kernel, rows_per_blk=RB),
        grid=(T // RB,),
        in_specs=[
            pl.BlockSpec((RB, D), lambda i: (i, 0)),
            pl.BlockSpec((D, N_QUBITS), lambda i: (0, 0)),
            pl.BlockSpec((N_QUBITS, 1), lambda i: (0, 0)),
        ],
        out_specs=[
            pl.BlockSpec((N_BLOCKS, N_BLOCKS), lambda i: (0, 0)),
            pl.BlockSpec((N_BLOCKS, D), lambda i: (0, 0)),
        ],
        out_shape=[
            jax.ShapeDtypeStruct((N_BLOCKS, N_BLOCKS), jnp.int32),
            jax.ShapeDtypeStruct((N_BLOCKS, D), jnp.float32),
        ],
    )(xin, W_addr, b_addr.reshape(N_QUBITS, 1))
    lw = lw8[:, 0]

    aw_rt = pl.pallas_call(
        functools.partial(_read_kernel, rows_per_blk=RB),
        grid=(T // RB,),
        in_specs=[
            pl.BlockSpec((RB, D), lambda i: (i, 0)),
            pl.BlockSpec((D, N_QUBITS), lambda i: (0, 0)),
            pl.BlockSpec((N_QUBITS, 1), lambda i: (0, 0)),
        ],
        out_specs=pl.BlockSpec((N_BLOCKS, RB), lambda i: (0, i)),
        out_shape=jax.ShapeDtypeStruct((N_BLOCKS, T), jnp.float32),
    )(xq, W_addr, b_addr.reshape(N_QUBITS, 1))

    G, M8, G8 = pl.pallas_call(
        _factor_kernel,
        grid=(1,),
        in_specs=[
            pl.BlockSpec(memory_space=pltpu.SMEM),
            pl.BlockSpec((N_BLOCKS, D), lambda i: (0, 0)),
            pl.BlockSpec((N_BLOCKS, MEM_SIZE), lambda i: (0, 0)),
            pl.BlockSpec((D, MEM_SIZE), lambda i: (0, 0)),
            pl.BlockSpec((1, MEM_SIZE), lambda i: (0, 0)),
            pl.BlockSpec((MEM_SIZE, D), lambda i: (0, 0)),
            pl.BlockSpec((1, D), lambda i: (0, 0)),
            pl.BlockSpec((D, D), lambda i: (0, 0)),
            pl.BlockSpec((D, D), lambda i: (0, 0)),
            pl.BlockSpec((D, D), lambda i: (0, 0)),
            pl.BlockSpec((D, D), lambda i: (0, 0)),
            pl.BlockSpec((D, D), lambda i: (0, 0)),
        ],
        out_specs=[
            pl.BlockSpec((N_BLOCKS, N_BLOCKS), lambda i: (0, 0)),
            pl.BlockSpec((N_BLOCKS, D), lambda i: (0, 0)),
            pl.BlockSpec((N_BLOCKS, D), lambda i: (0, 0)),
        ],
        out_shape=[
            jax.ShapeDtypeStruct((N_BLOCKS, N_BLOCKS), jnp.float32),
            jax.ShapeDtypeStruct((N_BLOCKS, D), jnp.float32),
            jax.ShapeDtypeStruct((N_BLOCKS, D), jnp.float32),
        ],
    )(lw, x8, memory_data, W_content, b_content.reshape(1, MEM_SIZE),
      W_out, b_out.reshape(1, D), Wq, Wk, Wv, Wo, W_gate)

    wg2_bf = W_gate[D:].astype(jnp.bfloat16)
    BS = 512
    n_s = S // BS
    out = pl.pallas_call(
        _attn_kernel,
        grid=(B, n_s),
        in_specs=[
            pl.BlockSpec((N_BLOCKS, S), lambda b, s: (0, b)),
            pl.BlockSpec((N_BLOCKS, BS), lambda b, s: (0, b * n_s + s)),
            pl.BlockSpec((N_BLOCKS, N_BLOCKS), lambda b, s: (0, 0)),
            pl.BlockSpec((N_BLOCKS, D), lambda b, s: (0, 0)),
            pl.BlockSpec((N_BLOCKS, D), lambda b, s: (0, 0)),
            pl.BlockSpec((1, BS, D), lambda b, s: (b, s, 0)),
            pl.BlockSpec((D, D), lambda b, s: (0, 0)),
            pl.BlockSpec((1, D), lambda b, s: (0, 0)),
        ],
        out_specs=pl.BlockSpec((1, BS, D), lambda b, s: (b, s, 0)),
        out_shape=jax.ShapeDtypeStruct((B, S, D), jnp.float32),
    )(aw_rt, aw_rt, G, M8, G8, input_data, wg2_bf, b_gate.reshape(1, D))
    return out
